# baseline (speedup 1.0000x reference)
import functools

import jax
import jax.numpy as jnp
from jax import lax
from jax.experimental import pallas as pl
from jax.experimental.pallas import tpu as pltpu
from jax.experimental.pallas import tpu_sc as plsc

_G = 128
_TOTAL = _G * _G * _G
_HALF = _TOTAL // 2
_TRASH = 1024
_NSUB = 16
_SLICE = _HALF // _NSUB

_CLS_VALUES = (0, 2, 3, 4, 5)
_CLS_CH = (8192, 8192, 8192, 8192, 8192)
_CLS_CHUNKS = (12, 1, 1, 1, 1)
_CLS_PAD = tuple(ch * c * _NSUB for ch, c in zip(_CLS_CH, _CLS_CHUNKS))
_CLS_VOFF = (0, 8192, 16384, 24576, 32768)


def _sc_scatter(cell_i, walls_i, inlets_i, outlets_i, empties_i, ones_hbm,
                vals_hbm):
    mesh = plsc.VectorSubcoreMesh(core_axis_name="c", subcore_axis_name="s")

    @functools.partial(
        pl.kernel,
        mesh=mesh,
        out_type=jax.ShapeDtypeStruct((_TOTAL,), jnp.int32),
        scratch_types=[
            pltpu.VMEM((64, 128), jnp.int32),
            pltpu.VMEM((64, 128), jnp.int32),
            pltpu.VMEM((8192,), jnp.int32),
            pltpu.VMEM((8192,), jnp.int32),
            pltpu.VMEM((8192,), jnp.int32),
            pltpu.VMEM((8192,), jnp.int32),
            pltpu.VMEM_SHARED((_HALF + _TRASH,), jnp.int32),
            pltpu.SemaphoreType.DMA,
            pltpu.SemaphoreType.DMA,
            pltpu.SemaphoreType.DMA,
            pltpu.SemaphoreType.DMA,
        ],
    )
    def body(cell_h, walls_h, inlets_h, outlets_h, empties_h, ones_h, vals_h,
             out_h, r0, r1, i0, i1, vA, vB, spmem,
             ls0, ls1, ss0, ss1):
        c = lax.axis_index("c")
        s = lax.axis_index("s")
        sc_base = c * _HALF

        def route(raw_b, idx_b, chr_):
            def row_body(r, _):
                for g in range(8):
                    v = raw_b[r, pl.ds(g * 16, 16)]
                    local = v - sc_base
                    valid = (local >= 0) & (local < _HALF)
                    trash = _HALF + (v & (_TRASH - 1))
                    idx_b[pl.ds(r * 128 + g * 16, 16)] = jnp.where(
                        valid, local, trash)
                return 0

            lax.fori_loop(0, chr_, row_body, 0)

        n_cell = _CLS_CHUNKS[0]
        chr_a = _CLS_CH[0] // 128
        rbufs, ibufs = (r0, r1), (i0, i1)
        lsems, ssems = (ls0, ls1), (ss0, ss1)

        def cell_load(j):
            return pltpu.async_copy(
                cell_h.at[pl.ds((s * n_cell + j) * chr_a, chr_a)],
                rbufs[j % 2], lsems[j % 2])

        load0 = cell_load(0)
        pltpu.sync_copy(vals_h.at[pl.ds(_CLS_VOFF[0], _CLS_CH[0])], vA)
        pltpu.sync_copy(ones_h, spmem.at[pl.ds(s * _SLICE, _SLICE)])
        plsc.subcore_barrier()

        loads = [load0]
        scats = []
        for j in range(n_cell):
            loads[j].wait()
            if j + 1 < n_cell:
                loads.append(cell_load(j + 1))
            route(rbufs[j % 2], ibufs[j % 2], chr_a)
            if j >= 1:
                scats[j - 1].wait()
            scats.append(pltpu.async_copy(vA, spmem.at[ibufs[j % 2]],
                                          ssems[j % 2]))
        scats[-1].wait()

        pltpu.async_copy(walls_h.at[pl.ds(s * 64, 64)], r0, ls0).wait()
        route(r0, i0, 64)
        pltpu.sync_copy(vals_h.at[pl.ds(_CLS_VOFF[1], 8192)], vA)
        pltpu.async_copy(inlets_h.at[pl.ds(s * 64, 64)], r1, ls1).wait()
        route(r1, i1, 64)
        pltpu.sync_copy(vals_h.at[pl.ds(_CLS_VOFF[2], 8192)], vB)

        plsc.subcore_barrier()
        pltpu.sync_copy(vA, spmem.at[i0])
        plsc.subcore_barrier()
        pltpu.async_copy(outlets_h.at[pl.ds(s * 64, 64)], r0, ls0).wait()
        route(r0, i0, 64)
        pltpu.sync_copy(vals_h.at[pl.ds(_CLS_VOFF[3], 8192)], vA)
        pltpu.sync_copy(vB, spmem.at[i1])
        plsc.subcore_barrier()
        pltpu.async_copy(empties_h.at[pl.ds(s * 64, 64)], r1, ls1).wait()
        route(r1, i1, 64)
        pltpu.sync_copy(vals_h.at[pl.ds(_CLS_VOFF[4], 8192)], vB)
        pltpu.sync_copy(vA, spmem.at[i0])
        plsc.subcore_barrier()
        pltpu.sync_copy(vB, spmem.at[i1])
        plsc.subcore_barrier()

        pltpu.sync_copy(spmem.at[pl.ds(s * _SLICE, _SLICE)],
                        out_h.at[pl.ds(sc_base + s * _SLICE, _SLICE)])

    return body(cell_i, walls_i, inlets_i, outlets_i, empties_i, ones_hbm,
                vals_hbm)


_BLK = 32768
_NBLK = _TOTAL // _BLK


_BROWS = _BLK // 128


def _expand_body(ct_ref, emb_ref, out_ref):
    t = ct_ref[...].reshape(1, _BROWS, 128)
    tb = jnp.broadcast_to(t, (16, _BROWS, 128))
    acc = jnp.broadcast_to(emb_ref[:, 5].reshape(16, 1, 1),
                           (16, _BROWS, 128))
    for k in (4, 3, 2, 1, 0):
        acc = jnp.where(tb == k,
                        jnp.broadcast_to(emb_ref[:, k].reshape(16, 1, 1),
                                         (16, _BROWS, 128)),
                        acc)
    out_ref[...] = acc


def _tc_expand(ct, emb_t8):
    ct2 = ct.reshape(_TOTAL // 128, 128)
    out = pl.pallas_call(
        _expand_body,
        grid=(_NBLK,),
        in_specs=[
            pl.BlockSpec((_BROWS, 128), lambda i: (i, 0)),
            pl.BlockSpec((16, 8), lambda i: (0, 0)),
        ],
        out_specs=pl.BlockSpec((16, _BROWS, 128), lambda i: (0, i, 0)),
        out_shape=jax.ShapeDtypeStruct((16, _TOTAL // 128, 128), jnp.float32),
    )(ct2, emb_t8)
    return out.reshape(16, _G, _G, _G)


def _pad2d(idx, padded):
    n = idx.shape[0]
    return jnp.concatenate(
        [idx.astype(jnp.int32),
         jnp.full((padded - n,), _TOTAL, jnp.int32)]).reshape(padded // 128,
                                                             128)


def kernel(cell_idx, walls_idx, inlets_idx, outlets_idx, empties_idx,
           embedding):
    arrs = (cell_idx, walls_idx, inlets_idx, outlets_idx, empties_idx)
    padded = [_pad2d(a, p) for a, p in zip(arrs, _CLS_PAD)]
    ones = jnp.ones((_SLICE,), jnp.int32)
    vals = jnp.concatenate([
        jnp.full((8192,), 0, jnp.int32),
        jnp.full((8192,), 2, jnp.int32),
        jnp.full((8192,), 3, jnp.int32),
        jnp.full((8192,), 4, jnp.int32),
        jnp.full((8192,), 5, jnp.int32),
    ])
    ct = _sc_scatter(*padded, ones, vals)
    emb_t8 = jnp.zeros((16, 8), jnp.float32).at[:, :6].set(embedding.T)
    return _tc_expand(ct, emb_t8)


# speedup vs baseline: 2.2801x; 2.2801x over previous
import functools

import jax
import jax.numpy as jnp
from jax import lax
from jax.experimental import pallas as pl
from jax.experimental.pallas import tpu as pltpu
from jax.experimental.pallas import tpu_sc as plsc

_G = 128
_TOTAL = _G * _G * _G
_HALF = _TOTAL // 2
_TRASH = 1024
_NSUB = 16
_SLICE = _HALF // _NSUB

_CLS_VALUES = (0, 2, 3, 4, 5)
_CLS_CH = (8192, 8192, 8192, 8192, 8192)
_CLS_CHUNKS = (12, 1, 1, 1, 1)
_CLS_PAD = tuple(ch * c * _NSUB for ch, c in zip(_CLS_CH, _CLS_CHUNKS))
_CLS_VOFF = (0, 8192, 16384, 24576, 32768)


def _sc_scatter(cell_i, walls_i, inlets_i, outlets_i, empties_i, ones_hbm,
                vals_hbm):
    mesh = plsc.VectorSubcoreMesh(core_axis_name="c", subcore_axis_name="s")

    @functools.partial(
        pl.kernel,
        mesh=mesh,
        out_type=jax.ShapeDtypeStruct((_TOTAL,), jnp.int32),
        scratch_types=[
            pltpu.VMEM((64, 128), jnp.int32),
            pltpu.VMEM((64, 128), jnp.int32),
            pltpu.VMEM((8192,), jnp.int32),
            pltpu.VMEM((8192,), jnp.int32),
            pltpu.VMEM((8192,), jnp.int32),
            pltpu.VMEM((8192,), jnp.int32),
            pltpu.VMEM_SHARED((_HALF + _TRASH,), jnp.int32),
            pltpu.SemaphoreType.DMA,
            pltpu.SemaphoreType.DMA,
            pltpu.SemaphoreType.DMA,
            pltpu.SemaphoreType.DMA,
        ],
    )
    def body(cell_h, walls_h, inlets_h, outlets_h, empties_h, ones_h, vals_h,
             out_h, r0, r1, i0, i1, vA, vB, spmem,
             ls0, ls1, ss0, ss1):
        c = lax.axis_index("c")
        s = lax.axis_index("s")
        sc_base = c * _HALF

        lane = lax.iota(jnp.int32, 16)
        trash_g = [_HALF + s * 64 + (g % 4) * 16 + lane for g in range(8)]

        def route(raw_b, idx_b, chr_):
            def row_body(r, _):
                for g in range(8):
                    v = raw_b[r, pl.ds(g * 16, 16)]
                    local = v - sc_base
                    valid = (local >= 0) & (local < _HALF)
                    idx_b[pl.ds(r * 128 + g * 16, 16)] = jnp.where(
                        valid, local, trash_g[g])
                return 0

            lax.fori_loop(0, chr_, row_body, 0)

        n_cell = _CLS_CHUNKS[0]
        chr_a = _CLS_CH[0] // 128
        rbufs, ibufs = (r0, r1), (i0, i1)
        lsems, ssems = (ls0, ls1), (ss0, ss1)

        def cell_load(j):
            return pltpu.async_copy(
                cell_h.at[pl.ds((s * n_cell + j) * chr_a, chr_a)],
                rbufs[j % 2], lsems[j % 2])

        load0 = cell_load(0)
        pltpu.sync_copy(vals_h.at[pl.ds(_CLS_VOFF[0], _CLS_CH[0])], vA)
        pltpu.sync_copy(ones_h, spmem.at[pl.ds(s * _SLICE, _SLICE)])
        plsc.subcore_barrier()

        loads = [load0]
        scats = []
        for j in range(n_cell):
            loads[j].wait()
            if j + 1 < n_cell:
                loads.append(cell_load(j + 1))
            route(rbufs[j % 2], ibufs[j % 2], chr_a)
            if j >= 1:
                scats[j - 1].wait()
            scats.append(pltpu.async_copy(vA, spmem.at[ibufs[j % 2]],
                                          ssems[j % 2]))
        scats[-1].wait()

        pltpu.async_copy(walls_h.at[pl.ds(s * 64, 64)], r0, ls0).wait()
        route(r0, i0, 64)
        pltpu.sync_copy(vals_h.at[pl.ds(_CLS_VOFF[1], 8192)], vA)
        pltpu.async_copy(inlets_h.at[pl.ds(s * 64, 64)], r1, ls1).wait()
        route(r1, i1, 64)
        pltpu.sync_copy(vals_h.at[pl.ds(_CLS_VOFF[2], 8192)], vB)

        plsc.subcore_barrier()
        pltpu.sync_copy(vA, spmem.at[i0])
        plsc.subcore_barrier()
        pltpu.async_copy(outlets_h.at[pl.ds(s * 64, 64)], r0, ls0).wait()
        route(r0, i0, 64)
        pltpu.sync_copy(vals_h.at[pl.ds(_CLS_VOFF[3], 8192)], vA)
        pltpu.sync_copy(vB, spmem.at[i1])
        plsc.subcore_barrier()
        pltpu.async_copy(empties_h.at[pl.ds(s * 64, 64)], r1, ls1).wait()
        route(r1, i1, 64)
        pltpu.sync_copy(vals_h.at[pl.ds(_CLS_VOFF[4], 8192)], vB)
        pltpu.sync_copy(vA, spmem.at[i0])
        plsc.subcore_barrier()
        pltpu.sync_copy(vB, spmem.at[i1])
        plsc.subcore_barrier()

        pltpu.sync_copy(spmem.at[pl.ds(s * _SLICE, _SLICE)],
                        out_h.at[pl.ds(sc_base + s * _SLICE, _SLICE)])

    return body(cell_i, walls_i, inlets_i, outlets_i, empties_i, ones_hbm,
                vals_hbm)


_BLK = 32768
_NBLK = _TOTAL // _BLK


_BROWS = _BLK // 128


def _expand_body(ct_ref, emb_ref, out_ref):
    t = ct_ref[...].reshape(1, _BROWS, 128)
    tb = jnp.broadcast_to(t, (16, _BROWS, 128))
    acc = jnp.broadcast_to(emb_ref[:, 5].reshape(16, 1, 1),
                           (16, _BROWS, 128))
    for k in (4, 3, 2, 1, 0):
        acc = jnp.where(tb == k,
                        jnp.broadcast_to(emb_ref[:, k].reshape(16, 1, 1),
                                         (16, _BROWS, 128)),
                        acc)
    out_ref[...] = acc


def _tc_expand(ct, emb_t8):
    ct2 = ct.reshape(_TOTAL // 128, 128)
    out = pl.pallas_call(
        _expand_body,
        grid=(_NBLK,),
        in_specs=[
            pl.BlockSpec((_BROWS, 128), lambda i: (i, 0)),
            pl.BlockSpec((16, 8), lambda i: (0, 0)),
        ],
        out_specs=pl.BlockSpec((16, _BROWS, 128), lambda i: (0, i, 0)),
        out_shape=jax.ShapeDtypeStruct((16, _TOTAL // 128, 128), jnp.float32),
    )(ct2, emb_t8)
    return out.reshape(16, _G, _G, _G)


def _pad2d(idx, padded):
    n = idx.shape[0]
    return jnp.concatenate(
        [idx.astype(jnp.int32),
         jnp.full((padded - n,), _TOTAL, jnp.int32)]).reshape(padded // 128,
                                                             128)


def kernel(cell_idx, walls_idx, inlets_idx, outlets_idx, empties_idx,
           embedding):
    arrs = (cell_idx, walls_idx, inlets_idx, outlets_idx, empties_idx)
    padded = [_pad2d(a, p) for a, p in zip(arrs, _CLS_PAD)]
    ones = jnp.ones((_SLICE,), jnp.int32)
    vals = jnp.concatenate([
        jnp.full((8192,), 0, jnp.int32),
        jnp.full((8192,), 2, jnp.int32),
        jnp.full((8192,), 3, jnp.int32),
        jnp.full((8192,), 4, jnp.int32),
        jnp.full((8192,), 5, jnp.int32),
    ])
    ct = _sc_scatter(*padded, ones, vals)
    emb_t8 = jnp.zeros((16, 8), jnp.float32).at[:, :6].set(embedding.T)
    return _tc_expand(ct, emb_t8)


# speedup vs baseline: 2.5065x; 1.0993x over previous
import functools

import jax
import jax.numpy as jnp
from jax import lax
from jax.experimental import pallas as pl
from jax.experimental.pallas import tpu as pltpu
from jax.experimental.pallas import tpu_sc as plsc

_G = 128
_TOTAL = _G * _G * _G
_HALF = _TOTAL // 2
_TRASH = 1024
_NSUB = 16
_SLICE = _HALF // _NSUB

_CLS_VALUES = (0, 2, 3, 4, 5)
_CLS_CH = (8192, 8192, 8192, 8192, 8192)
_CLS_CHUNKS = (12, 1, 1, 1, 1)
_CLS_PAD = tuple(ch * c * _NSUB for ch, c in zip(_CLS_CH, _CLS_CHUNKS))
_CLS_VOFF = (0, 8192, 16384, 24576, 32768)


def _sc_scatter(cell_i, walls_i, inlets_i, outlets_i, empties_i, ones_hbm,
                vals_hbm):
    mesh = plsc.VectorSubcoreMesh(core_axis_name="c", subcore_axis_name="s")

    @functools.partial(
        pl.kernel,
        mesh=mesh,
        out_type=jax.ShapeDtypeStruct((_TOTAL,), jnp.int32),
        scratch_types=[
            pltpu.VMEM((64, 128), jnp.int32),
            pltpu.VMEM((64, 128), jnp.int32),
            pltpu.VMEM((8192,), jnp.int32),
            pltpu.VMEM((8192,), jnp.int32),
            pltpu.VMEM((8192,), jnp.int32),
            pltpu.VMEM((8192,), jnp.int32),
            pltpu.VMEM_SHARED((_HALF + _TRASH,), jnp.int32),
            pltpu.SemaphoreType.DMA,
            pltpu.SemaphoreType.DMA,
            pltpu.SemaphoreType.DMA,
            pltpu.SemaphoreType.DMA,
        ],
    )
    def body(cell_h, walls_h, inlets_h, outlets_h, empties_h, ones_h, vals_h,
             out_h, r0, r1, i0, i1, vA, vB, spmem,
             ls0, ls1, ss0, ss1):
        c = lax.axis_index("c")
        s = lax.axis_index("s")
        sc_base = c * _HALF

        lane = lax.iota(jnp.int32, 16)
        trash_g = [_HALF + s * 64 + (g % 4) * 16 + lane for g in range(8)]

        def route(raw_b, idx_b, chr_):
            def row_body(r, _):
                for g in range(8):
                    v = raw_b[r, pl.ds(g * 16, 16)]
                    local = v - sc_base
                    valid = (local >= 0) & (local < _HALF)
                    idx_b[pl.ds(r * 128 + g * 16, 16)] = jnp.where(
                        valid, local, trash_g[g])
                return 0

            lax.fori_loop(0, chr_, row_body, 0)

        n_cell = _CLS_CHUNKS[0]
        chr_a = _CLS_CH[0] // 128
        rbufs, ibufs = (r0, r1), (i0, i1)
        lsems, ssems = (ls0, ls1), (ss0, ss1)

        def cell_load(j):
            return pltpu.async_copy(
                cell_h.at[pl.ds((s * n_cell + j) * chr_a, chr_a)],
                rbufs[j % 2], lsems[j % 2])

        load0 = cell_load(0)
        pltpu.sync_copy(vals_h.at[pl.ds(_CLS_VOFF[0], _CLS_CH[0])], vA)
        pltpu.sync_copy(ones_h, spmem.at[pl.ds(s * _SLICE, _SLICE)])
        plsc.subcore_barrier()

        loads = [load0]
        scats = []
        for j in range(n_cell):
            loads[j].wait()
            if j + 1 < n_cell:
                loads.append(cell_load(j + 1))
            route(rbufs[j % 2], ibufs[j % 2], chr_a)
            if j >= 1:
                scats[j - 1].wait()
            scats.append(pltpu.async_copy(vA, spmem.at[ibufs[j % 2]],
                                          ssems[j % 2]))
        scats[-1].wait()

        pltpu.async_copy(walls_h.at[pl.ds(s * 64, 64)], r0, ls0).wait()
        route(r0, i0, 64)
        pltpu.sync_copy(vals_h.at[pl.ds(_CLS_VOFF[1], 8192)], vA)
        pltpu.async_copy(inlets_h.at[pl.ds(s * 64, 64)], r1, ls1).wait()
        route(r1, i1, 64)
        pltpu.sync_copy(vals_h.at[pl.ds(_CLS_VOFF[2], 8192)], vB)

        plsc.subcore_barrier()
        pltpu.sync_copy(vA, spmem.at[i0])
        plsc.subcore_barrier()
        pltpu.async_copy(outlets_h.at[pl.ds(s * 64, 64)], r0, ls0).wait()
        route(r0, i0, 64)
        pltpu.sync_copy(vals_h.at[pl.ds(_CLS_VOFF[3], 8192)], vA)
        pltpu.sync_copy(vB, spmem.at[i1])
        plsc.subcore_barrier()
        pltpu.async_copy(empties_h.at[pl.ds(s * 64, 64)], r1, ls1).wait()
        route(r1, i1, 64)
        pltpu.sync_copy(vals_h.at[pl.ds(_CLS_VOFF[4], 8192)], vB)
        pltpu.sync_copy(vA, spmem.at[i0])
        plsc.subcore_barrier()
        pltpu.sync_copy(vB, spmem.at[i1])
        plsc.subcore_barrier()

        pltpu.sync_copy(spmem.at[pl.ds(s * _SLICE, _SLICE)],
                        out_h.at[pl.ds(sc_base + s * _SLICE, _SLICE)])

    return body(cell_i, walls_i, inlets_i, outlets_i, empties_i, ones_hbm,
                vals_hbm)


_BLK = 65536
_NBLK = _TOTAL // _BLK


_BROWS = _BLK // 128


def _expand_body(ct_ref, emb_ref, out_ref):
    t = ct_ref[...].reshape(1, _BROWS, 128)
    tb = jnp.broadcast_to(t, (16, _BROWS, 128))
    acc = jnp.broadcast_to(emb_ref[:, 5].reshape(16, 1, 1),
                           (16, _BROWS, 128))
    for k in (4, 3, 2, 1, 0):
        acc = jnp.where(tb == k,
                        jnp.broadcast_to(emb_ref[:, k].reshape(16, 1, 1),
                                         (16, _BROWS, 128)),
                        acc)
    out_ref[...] = acc


def _tc_expand(ct, emb_t8):
    ct2 = ct.reshape(_TOTAL // 128, 128)
    out = pl.pallas_call(
        _expand_body,
        grid=(_NBLK,),
        in_specs=[
            pl.BlockSpec((_BROWS, 128), lambda i: (i, 0)),
            pl.BlockSpec((16, 8), lambda i: (0, 0)),
        ],
        out_specs=pl.BlockSpec((16, _BROWS, 128), lambda i: (0, i, 0)),
        out_shape=jax.ShapeDtypeStruct((16, _TOTAL // 128, 128), jnp.float32),
    )(ct2, emb_t8)
    return out.reshape(16, _G, _G, _G)


def _pad2d(idx, padded):
    n = idx.shape[0]
    return jnp.concatenate(
        [idx.astype(jnp.int32),
         jnp.full((padded - n,), _TOTAL, jnp.int32)]).reshape(padded // 128,
                                                             128)


def kernel(cell_idx, walls_idx, inlets_idx, outlets_idx, empties_idx,
           embedding):
    arrs = (cell_idx, walls_idx, inlets_idx, outlets_idx, empties_idx)
    padded = [_pad2d(a, p) for a, p in zip(arrs, _CLS_PAD)]
    ones = jnp.ones((_SLICE,), jnp.int32)
    vals = jnp.concatenate([
        jnp.full((8192,), 0, jnp.int32),
        jnp.full((8192,), 2, jnp.int32),
        jnp.full((8192,), 3, jnp.int32),
        jnp.full((8192,), 4, jnp.int32),
        jnp.full((8192,), 5, jnp.int32),
    ])
    ct = _sc_scatter(*padded, ones, vals)
    emb_t8 = jnp.zeros((16, 8), jnp.float32).at[:, :6].set(embedding.T)
    return _tc_expand(ct, emb_t8)
